# Initial kernel scaffold; baseline (speedup 1.0000x reference)
import jax
import jax.numpy as jnp
from jax.experimental import pallas as pl


def kernel(x, Wih, Whh, b_lstm, K, X, Ws0, Wm0, bl0, Ws1, Wm1, bl1, hW1, hb1, hW2, hb2, edge_index):
    raise NotImplementedError("write your pallas kernel here")


# speedup vs baseline: 2.4545x; 2.4545x over previous
import functools

import jax
import jax.numpy as jnp
from jax.experimental import pallas as pl

DT = 1.0


def _fused(B, N, T, F, H, HOR, E,
           xT_ref, Wih_ref, Whh_ref, b_ref, K_ref, X_ref, src_ref, dst_ref,
           Ws0_ref, Wm0_ref, bl0_ref, Ws1_ref, Wm1_ref, bl1_ref,
           hW1_ref, hb1_ref, hW2_ref, hb2_ref, out_ref):
    BN = B * N
    Wih = Wih_ref[...]
    Whh = Whh_ref[...]
    b = b_ref[...]

    def step(t, carry):
        h, c = carry
        xt = xT_ref[t]
        gates = (jnp.dot(xt, Wih, preferred_element_type=jnp.float32)
                 + jnp.dot(h, Whh, preferred_element_type=jnp.float32) + b)
        i = jax.nn.sigmoid(gates[:, 0:H])
        f = jax.nn.sigmoid(gates[:, H:2 * H])
        g = jnp.tanh(gates[:, 2 * H:3 * H])
        o = jax.nn.sigmoid(gates[:, 3 * H:4 * H])
        c = f * c + i * g
        h = o * jnp.tanh(c)
        return (h, c)

    h0 = jnp.zeros((BN, H), jnp.float32)
    c0 = jnp.zeros((BN, H), jnp.float32)
    h, _ = jax.lax.fori_loop(0, T, step, (h0, c0))

    K = K_ref[...]
    X = X_ref[...]
    denom = K - K * X + 0.5 * DT
    w = ((-K * X + 0.5 * DT) / denom) + ((K * X + 0.5 * DT) / denom)
    node_ids = jax.lax.broadcasted_iota(jnp.int32, (N, E), 0)
    src_oh = src_ref[...] == node_ids
    dst_oh = dst_ref[...] == node_ids
    wsrc = jnp.where(src_oh, w, 0.0)
    dstf = jnp.where(dst_oh, 1.0, 0.0)
    A = jax.lax.dot_general(dstf, wsrc, (((1,), (1,)), ((), ())),
                            preferred_element_type=jnp.float32)

    def graph_layer(hcur, Ws, Wm, bl):
        hWs = jnp.dot(hcur, Ws, preferred_element_type=jnp.float32)
        aggs = []
        for bi in range(B):
            hb = hcur[bi * N:(bi + 1) * N]
            aggs.append(jnp.dot(A, hb, preferred_element_type=jnp.float32))
        agg = jnp.concatenate(aggs, axis=0)
        z = hWs + jnp.dot(agg, Wm, preferred_element_type=jnp.float32) + bl
        return jax.nn.gelu(z) + hcur

    h = graph_layer(h, Ws0_ref[...], Wm0_ref[...], bl0_ref[...])
    h = graph_layer(h, Ws1_ref[...], Wm1_ref[...], bl1_ref[...])

    z = jax.nn.gelu(jnp.dot(h, hW1_ref[...], preferred_element_type=jnp.float32)
                    + hb1_ref[...])
    out_ref[...] = (jnp.dot(z, hW2_ref[...], preferred_element_type=jnp.float32)
                    + hb2_ref[...])


def kernel(x, Wih, Whh, b_lstm, K, X, Ws0, Wm0, bl0, Ws1, Wm1, bl1,
           hW1, hb1, hW2, hb2, edge_index):
    B, N, T, F = x.shape
    H = Whh.shape[0]
    HOR = hW2.shape[1]
    E = edge_index.shape[1]
    BN = B * N

    xT = jnp.transpose(x.reshape(BN, T, F), (1, 0, 2))
    src = edge_index[0:1, :]
    dst = edge_index[1:2, :]
    args = (xT, Wih, Whh, b_lstm.reshape(1, 4 * H), K.reshape(1, E),
            X.reshape(1, E), src, dst, Ws0, Wm0, bl0.reshape(1, H),
            Ws1, Wm1, bl1.reshape(1, H), hW1, hb1.reshape(1, H),
            hW2, hb2.reshape(1, HOR))

    out = pl.pallas_call(
        functools.partial(_fused, B, N, T, F, H, HOR, E),
        out_shape=jax.ShapeDtypeStruct((BN, HOR), jnp.float32),
    )(*args)
    return out.reshape(B, N, HOR)


# speedup vs baseline: 2.5439x; 1.0364x over previous
import functools

import jax
import jax.numpy as jnp
from jax.experimental import pallas as pl

DT = 1.0


def _fused(B, N, T, F, H, HOR, E,
           xT_ref, Wih_ref, Whh_ref, b_ref, K_ref, X_ref, src_ref, dst_ref,
           Ws0_ref, Wm0_ref, bl0_ref, Ws1_ref, Wm1_ref, bl1_ref,
           hW1_ref, hb1_ref, hW2_ref, hb2_ref, out_ref):
    BN = B * N
    Wih = Wih_ref[...]
    Whh = Whh_ref[...]
    b = b_ref[...]

    C = 8
    CH = T // C

    def sigm(v):
        return 0.5 * jnp.tanh(0.5 * v) + 0.5

    def proj(ci):
        xc = xT_ref[pl.ds(ci * C, C)].reshape(C * BN, F)
        return jnp.dot(xc, Wih, preferred_element_type=jnp.float32) + b

    def outer(ci, carry):
        h, c, XW = carry
        XW_next = proj(jnp.minimum(ci + 1, CH - 1))
        for k in range(C):
            gates = XW[k * BN:(k + 1) * BN] + jnp.dot(
                h, Whh, preferred_element_type=jnp.float32)
            i = sigm(gates[:, 0:H])
            f = sigm(gates[:, H:2 * H])
            g = jnp.tanh(gates[:, 2 * H:3 * H])
            o = sigm(gates[:, 3 * H:4 * H])
            c = f * c + i * g
            h = o * jnp.tanh(c)
        return (h, c, XW_next)

    h0 = jnp.zeros((BN, H), jnp.float32)
    c0 = jnp.zeros((BN, H), jnp.float32)
    h, _, _ = jax.lax.fori_loop(0, CH, outer, (h0, c0, proj(0)))

    K = K_ref[...]
    X = X_ref[...]
    denom = K - K * X + 0.5 * DT
    w = ((-K * X + 0.5 * DT) / denom) + ((K * X + 0.5 * DT) / denom)
    node_ids = jax.lax.broadcasted_iota(jnp.int32, (N, E), 0)
    src_oh = src_ref[...] == node_ids
    dst_oh = dst_ref[...] == node_ids
    wsrc = jnp.where(src_oh, w, 0.0)
    dstf = jnp.where(dst_oh, 1.0, 0.0)
    A = jax.lax.dot_general(dstf, wsrc, (((1,), (1,)), ((), ())),
                            preferred_element_type=jnp.float32)

    def graph_layer(hcur, Ws, Wm, bl):
        hWs = jnp.dot(hcur, Ws, preferred_element_type=jnp.float32)
        aggs = []
        for bi in range(B):
            hb = hcur[bi * N:(bi + 1) * N]
            aggs.append(jnp.dot(A, hb, preferred_element_type=jnp.float32))
        agg = jnp.concatenate(aggs, axis=0)
        z = hWs + jnp.dot(agg, Wm, preferred_element_type=jnp.float32) + bl
        return jax.nn.gelu(z) + hcur

    h = graph_layer(h, Ws0_ref[...], Wm0_ref[...], bl0_ref[...])
    h = graph_layer(h, Ws1_ref[...], Wm1_ref[...], bl1_ref[...])

    z = jax.nn.gelu(jnp.dot(h, hW1_ref[...], preferred_element_type=jnp.float32)
                    + hb1_ref[...])
    out_ref[...] = (jnp.dot(z, hW2_ref[...], preferred_element_type=jnp.float32)
                    + hb2_ref[...])


def kernel(x, Wih, Whh, b_lstm, K, X, Ws0, Wm0, bl0, Ws1, Wm1, bl1,
           hW1, hb1, hW2, hb2, edge_index):
    B, N, T, F = x.shape
    H = Whh.shape[0]
    HOR = hW2.shape[1]
    E = edge_index.shape[1]
    BN = B * N

    xT = jnp.transpose(x.reshape(BN, T, F), (1, 0, 2))
    src = edge_index[0:1, :]
    dst = edge_index[1:2, :]
    args = (xT, Wih, Whh, b_lstm.reshape(1, 4 * H), K.reshape(1, E),
            X.reshape(1, E), src, dst, Ws0, Wm0, bl0.reshape(1, H),
            Ws1, Wm1, bl1.reshape(1, H), hW1, hb1.reshape(1, H),
            hW2, hb2.reshape(1, HOR))

    out = pl.pallas_call(
        functools.partial(_fused, B, N, T, F, H, HOR, E),
        out_shape=jax.ShapeDtypeStruct((BN, HOR), jnp.float32),
    )(*args)
    return out.reshape(B, N, HOR)
